# baseline (speedup 1.0000x reference)
import functools

import jax
import jax.numpy as jnp
from jax import lax
from jax.experimental import pallas as pl
from jax.experimental.pallas import tpu as pltpu
from jax.experimental.pallas import tpu_sc as plsc

N = 10000
E = 320000
E_HID = 128
T_HID = 64
R_HID = 64
NREL = 100
RP = 104
K = 128
NSC = 16
NCH = 160
EPAD = K * NCH * NSC
NGRP = 625
CROWS = RP * NGRP
CROWS_P = 65024
SLAB = CROWS_P // NSC
HROWS = 2000
BLK = 2560
OUT_W = R_HID + 2 * T_HID


def _h_body(x_ref, w_ref, b_ref, o_ref):
    o_ref[...] = lax.dot_general(
        x_ref[...], w_ref[...], (((1,), (1,)), ((), ())),
        preferred_element_type=jnp.float32) + b_ref[...]


def _project(x_e, W_tc1, b_tc1):
    return pl.pallas_call(
        _h_body,
        grid=(N // HROWS,),
        in_specs=[
            pl.BlockSpec((HROWS, E_HID), lambda i: (i, 0)),
            pl.BlockSpec((T_HID, E_HID), lambda i: (0, 0)),
            pl.BlockSpec((1, T_HID), lambda i: (0, 0)),
        ],
        out_specs=pl.BlockSpec((HROWS, T_HID), lambda i: (i, 0)),
        out_shape=jax.ShapeDtypeStruct((N, T_HID), jnp.float32),
    )(x_e, W_tc1, b_tc1.reshape(1, T_HID))


_SC_MESH = plsc.VectorSubcoreMesh(core_axis_name="c", subcore_axis_name="s")


@functools.partial(
    pl.kernel,
    mesh=_SC_MESH,
    compiler_params=pltpu.CompilerParams(use_tc_tiling_on_sc=False),
    out_type=jax.ShapeDtypeStruct((2, CROWS_P, 16), jnp.float32),
    scratch_types=[
        pltpu.VMEM((NCH, K), jnp.int32),
        pltpu.VMEM((NCH, K), jnp.int32),
        pltpu.VMEM((K,), jnp.int32),
        pltpu.VMEM((K,), jnp.int32),
        pltpu.VMEM((K, 16), jnp.float32),
        pltpu.VMEM_SHARED((CROWS_P, 16), jnp.float32),
        pltpu.SemaphoreType.DMA,
    ],
)
def _sc_histogram(headp, tailp, relp, zc, id16, oc,
                  nidx, ridx, nmidx, didx, oh_buf, hist, sem):
    cid = lax.axis_index("c")
    sid = lax.axis_index("s")

    pltpu.sync_copy(zc.at[pl.ds(sid * SLAB, SLAB)],
                    hist.at[pl.ds(sid * SLAB, SLAB)])

    base = sid * NCH

    @pl.when(cid == 0)
    def _load_heads():
        pltpu.sync_copy(headp.at[pl.ds(base, NCH)], nidx)

    @pl.when(cid == 1)
    def _load_tails():
        pltpu.sync_copy(tailp.at[pl.ds(base, NCH)], nidx)

    pltpu.sync_copy(relp.at[pl.ds(base, NCH)], ridx)
    plsc.subcore_barrier()

    def chunk(j, carry):
        def fill(s, c2):
            n = nidx[j, pl.ds(s * 16, 16)]
            r = ridx[j, pl.ds(s * 16, 16)]
            nmidx[pl.ds(s * 16, 16)] = lax.bitwise_and(n, 15)
            didx[pl.ds(s * 16, 16)] = (
                r * NGRP + lax.shift_right_logical(n, 4))
            return c2

        lax.fori_loop(0, K // 16, fill, 0)
        pltpu.async_copy(id16.at[nmidx], oh_buf, sem).wait()
        pltpu.sync_copy(oh_buf, hist.at[didx], add=True)
        return carry

    lax.fori_loop(0, NCH, chunk, 0)
    plsc.subcore_barrier()

    pltpu.sync_copy(hist.at[pl.ds(sid * SLAB, SLAB)],
                    oc.at[cid, pl.ds(sid * SLAB, SLAB)])


def _table_body(ch_ref, ct_ref, h_ref, w_ref, b_ref, tab_ref):
    ch = ch_ref[...]
    ct = ct_ref[...]
    h = h_ref[...]
    sum_h = lax.dot_general(ch, h, (((1,), (0,)), ((), ())),
                            preferred_element_type=jnp.float32)
    sum_t = lax.dot_general(ct, h, (((1,), (0,)), ((), ())),
                            preferred_element_type=jnp.float32)
    cnt = jnp.sum(ch, axis=1, keepdims=True)
    denom = jnp.maximum(cnt, 1.0)
    m = jnp.concatenate([sum_h / denom, sum_t / denom], axis=1)
    x2 = lax.dot_general(m, w_ref[...], (((1,), (1,)), ((), ())),
                         preferred_element_type=jnp.float32) + b_ref[...]
    tab_ref[...] = jnp.concatenate([x2, m], axis=1)


def _make_table(ch, ct, h, W_sr1, b_sr1):
    return pl.pallas_call(
        _table_body,
        out_shape=jax.ShapeDtypeStruct((RP, OUT_W), jnp.float32),
    )(ch, ct, h, W_sr1, b_sr1.reshape(1, R_HID))


def _out_body(rel_ref, xres_ref, tab_ref, o_ref):
    r = rel_ref[...]
    cols = lax.broadcasted_iota(jnp.int32, (1, RP), 1)
    onehot = jnp.equal(r, cols).astype(jnp.float32)
    look = lax.dot_general(
        onehot, tab_ref[...], (((1,), (0,)), ((), ())),
        preferred_element_type=jnp.float32)
    o_ref[:, 0:T_HID] = xres_ref[...] + look[:, 0:T_HID]
    o_ref[:, T_HID:] = look[:, T_HID:]


def _assemble(rel2, x_res1, table):
    return pl.pallas_call(
        _out_body,
        grid=(E // BLK,),
        in_specs=[
            pl.BlockSpec((BLK, 1), lambda i: (i, 0)),
            pl.BlockSpec((BLK, R_HID), lambda i: (i, 0)),
            pl.BlockSpec((RP, OUT_W), lambda i: (0, 0)),
        ],
        out_specs=pl.BlockSpec((BLK, OUT_W), lambda i: (i, 0)),
        out_shape=jax.ShapeDtypeStruct((E, OUT_W), jnp.float32),
        compiler_params=pltpu.CompilerParams(
            dimension_semantics=("arbitrary",)),
    )(rel2, x_res1, table)


def kernel(x_e, edge_index, rel, x_res1, rel_size, W_tc1, b_tc1, W_sr1,
           b_sr1, a1, a5):
    h = _project(x_e, W_tc1, b_tc1)
    pad = EPAD - E
    headp = jnp.concatenate(
        [edge_index[0], jnp.zeros((pad,), jnp.int32)]).reshape(-1, K)
    tailp = jnp.concatenate(
        [edge_index[1], jnp.zeros((pad,), jnp.int32)]).reshape(-1, K)
    relp = jnp.concatenate(
        [rel, jnp.full((pad,), NREL, jnp.int32)]).reshape(-1, K)
    zc = jnp.zeros((CROWS_P, 16), jnp.float32)
    id16 = jnp.eye(16, dtype=jnp.float32)
    oc = _sc_histogram(headp, tailp, relp, zc, id16)
    ch = oc[0, :CROWS, :].reshape(RP, N)
    ct = oc[1, :CROWS, :].reshape(RP, N)
    table = _make_table(ch, ct, h, W_sr1, b_sr1)
    return _assemble(rel.reshape(E, 1), x_res1, table)


# speedup vs baseline: 2.7605x; 2.7605x over previous
import functools

import jax
import jax.numpy as jnp
from jax import lax
from jax.experimental import pallas as pl
from jax.experimental.pallas import tpu as pltpu
from jax.experimental.pallas import tpu_sc as plsc

N = 10000
E = 320000
E_HID = 128
T_HID = 64
R_HID = 64
NREL = 100
RP = 104
NSC = 16
NPT = 625
CW = RP * NPT
CWP = 65024
ECH = 8192
EPAD = 327680
NCH = EPAD // ECH
HROWS = 2000
BLK = 2560
OUT_W = R_HID + 2 * T_HID


def _h_body(x_ref, w_ref, b_ref, o_ref):
    o_ref[...] = lax.dot_general(
        x_ref[...], w_ref[...], (((1,), (1,)), ((), ())),
        preferred_element_type=jnp.float32) + b_ref[...]


def _project(x_e, W_tc1, b_tc1):
    return pl.pallas_call(
        _h_body,
        grid=(N // HROWS,),
        in_specs=[
            pl.BlockSpec((HROWS, E_HID), lambda i: (i, 0)),
            pl.BlockSpec((T_HID, E_HID), lambda i: (0, 0)),
            pl.BlockSpec((1, T_HID), lambda i: (0, 0)),
        ],
        out_specs=pl.BlockSpec((HROWS, T_HID), lambda i: (i, 0)),
        out_shape=jax.ShapeDtypeStruct((N, T_HID), jnp.float32),
    )(x_e, W_tc1, b_tc1.reshape(1, T_HID))


_SC_MESH = plsc.VectorSubcoreMesh(core_axis_name="c", subcore_axis_name="s")


@functools.partial(
    pl.kernel,
    mesh=_SC_MESH,
    compiler_params=pltpu.CompilerParams(use_tc_tiling_on_sc=False,
                                         needs_layout_passes=False),
    out_type=jax.ShapeDtypeStruct((2, NSC, CWP), jnp.float32),
    scratch_types=[
        pltpu.VMEM((ECH,), jnp.int32),
        pltpu.VMEM((ECH,), jnp.int32),
        pltpu.VMEM((CWP,), jnp.float32),
    ],
)
def _sc_histogram(headp, tailp, relp, oc, nbuf, rbuf, ctab):
    cid = lax.axis_index("c")
    sid = lax.axis_index("s")
    lo = sid * NPT
    hi = lo + NPT
    ones16 = jnp.full((16,), 1.0, jnp.float32)
    zeros16 = jnp.zeros((16,), jnp.float32)

    def zero(i, carry):
        ctab[pl.ds(i * 16, 16)] = zeros16
        return carry

    lax.fori_loop(0, CWP // 16, zero, 0)

    def chunk(j, carry):
        off = j * ECH

        @pl.when(cid == 0)
        def _load_heads():
            pltpu.sync_copy(headp.at[pl.ds(off, ECH)], nbuf)

        @pl.when(cid == 1)
        def _load_tails():
            pltpu.sync_copy(tailp.at[pl.ds(off, ECH)], nbuf)

        pltpu.sync_copy(relp.at[pl.ds(off, ECH)], rbuf)

        def step(i, c2):
            for s in range(4):
                b = i * 64 + s * 16
                n = nbuf[pl.ds(b, 16)]
                r = rbuf[pl.ds(b, 16)]
                local = n - lo
                mask = jnp.logical_and(n >= lo, n < hi)
                addr = r * NPT + local
                plsc.addupdate_scatter(ctab, [addr], ones16, mask=mask)
            return c2

        lax.fori_loop(0, ECH // 64, step, 0)
        return carry

    lax.fori_loop(0, NCH, chunk, 0)
    pltpu.sync_copy(ctab, oc.at[cid, sid])


def _table_body(ch_ref, ct_ref, h_ref, w_ref, b_ref, tab_ref):
    ch = ch_ref[...]
    ct = ct_ref[...]
    h3 = h_ref[...]
    sum_h = jnp.sum(
        lax.dot_general(ch, h3, (((2,), (1,)), ((0,), (0,))),
                        preferred_element_type=jnp.float32), axis=0)
    sum_t = jnp.sum(
        lax.dot_general(ct, h3, (((2,), (1,)), ((0,), (0,))),
                        preferred_element_type=jnp.float32), axis=0)
    cnt = jnp.sum(jnp.sum(ch, axis=2, keepdims=True), axis=0)
    denom = jnp.maximum(cnt, 1.0)
    m = jnp.concatenate([sum_h / denom, sum_t / denom], axis=1)
    x2 = lax.dot_general(m, w_ref[...], (((1,), (1,)), ((), ())),
                         preferred_element_type=jnp.float32) + b_ref[...]
    tab_ref[...] = jnp.concatenate([x2, m], axis=1)


def _make_table(ch, ct, h, W_sr1, b_sr1):
    return pl.pallas_call(
        _table_body,
        out_shape=jax.ShapeDtypeStruct((RP, OUT_W), jnp.float32),
    )(ch, ct, h, W_sr1, b_sr1.reshape(1, R_HID))


def _out_body(rel_ref, xres_ref, tab_ref, o_ref):
    r = rel_ref[...]
    cols = lax.broadcasted_iota(jnp.int32, (1, RP), 1)
    onehot = jnp.equal(r, cols).astype(jnp.float32)
    look = lax.dot_general(
        onehot, tab_ref[...], (((1,), (0,)), ((), ())),
        preferred_element_type=jnp.float32)
    o_ref[:, 0:T_HID] = xres_ref[...] + look[:, 0:T_HID]
    o_ref[:, T_HID:] = look[:, T_HID:]


def _assemble(rel2, x_res1, table):
    return pl.pallas_call(
        _out_body,
        grid=(E // BLK,),
        in_specs=[
            pl.BlockSpec((BLK, 1), lambda i: (i, 0)),
            pl.BlockSpec((BLK, R_HID), lambda i: (i, 0)),
            pl.BlockSpec((RP, OUT_W), lambda i: (0, 0)),
        ],
        out_specs=pl.BlockSpec((BLK, OUT_W), lambda i: (i, 0)),
        out_shape=jax.ShapeDtypeStruct((E, OUT_W), jnp.float32),
        compiler_params=pltpu.CompilerParams(
            dimension_semantics=("arbitrary",)),
    )(rel2, x_res1, table)


def kernel(x_e, edge_index, rel, x_res1, rel_size, W_tc1, b_tc1, W_sr1,
           b_sr1, a1, a5):
    h = _project(x_e, W_tc1, b_tc1)
    pad = EPAD - E
    headf = jnp.concatenate([edge_index[0], jnp.zeros((pad,), jnp.int32)])
    tailf = jnp.concatenate([edge_index[1], jnp.zeros((pad,), jnp.int32)])
    relf = jnp.concatenate([rel, jnp.full((pad,), NREL, jnp.int32)])
    oc = _sc_histogram(headf, tailf, relf)
    ch = oc[0, :, :CW].reshape(NSC, RP, NPT)
    ct = oc[1, :, :CW].reshape(NSC, RP, NPT)
    h3 = h.reshape(NSC, NPT, T_HID)
    table = _make_table(ch, ct, h3, W_sr1, b_sr1)
    return _assemble(rel.reshape(E, 1), x_res1, table)


# speedup vs baseline: 2.7815x; 1.0076x over previous
import functools

import jax
import jax.numpy as jnp
from jax import lax
from jax.experimental import pallas as pl
from jax.experimental.pallas import tpu as pltpu
from jax.experimental.pallas import tpu_sc as plsc

N = 10000
E = 320000
E_HID = 128
T_HID = 64
R_HID = 64
NREL = 100
RP = 104
NSC = 16
NPT = 625
NPT_P = 640
ECH = 8192
EPAD = 327680
NCH = EPAD // ECH
HROWS = 2000
BLK = 2560
OUT_W = R_HID + 2 * T_HID


def _h_body(x_ref, w_ref, b_ref, o_ref):
    o_ref[...] = lax.dot_general(
        x_ref[...], w_ref[...], (((1,), (1,)), ((), ())),
        preferred_element_type=jnp.float32) + b_ref[...]


def _project(x_e, W_tc1, b_tc1):
    return pl.pallas_call(
        _h_body,
        grid=(N // HROWS,),
        in_specs=[
            pl.BlockSpec((HROWS, E_HID), lambda i: (i, 0)),
            pl.BlockSpec((T_HID, E_HID), lambda i: (0, 0)),
            pl.BlockSpec((1, T_HID), lambda i: (0, 0)),
        ],
        out_specs=pl.BlockSpec((HROWS, T_HID), lambda i: (i, 0)),
        out_shape=jax.ShapeDtypeStruct((N, T_HID), jnp.float32),
    )(x_e, W_tc1, b_tc1.reshape(1, T_HID))


_SC_MESH = plsc.VectorSubcoreMesh(core_axis_name="c", subcore_axis_name="s")


@functools.partial(
    pl.kernel,
    mesh=_SC_MESH,
    compiler_params=pltpu.CompilerParams(use_tc_tiling_on_sc=False,
                                         needs_layout_passes=False),
    out_type=jax.ShapeDtypeStruct((2, NSC, RP, NPT_P), jnp.float32),
    scratch_types=[
        pltpu.VMEM((ECH,), jnp.int32),
        pltpu.VMEM((ECH,), jnp.int32),
        pltpu.VMEM((RP, NPT_P), jnp.float32),
    ],
)
def _sc_histogram(headp, tailp, relp, zc, oc, nbuf, rbuf, ctab):
    cid = lax.axis_index("c")
    sid = lax.axis_index("s")
    lo = sid * NPT
    hi = lo + NPT

    ones16 = jnp.full((16,), 1.0, jnp.float32)
    pltpu.sync_copy(zc, ctab)

    def chunk(j, carry):
        off = j * ECH

        @pl.when(cid == 0)
        def _load_heads():
            pltpu.sync_copy(headp.at[pl.ds(off, ECH)], nbuf)

        @pl.when(cid == 1)
        def _load_tails():
            pltpu.sync_copy(tailp.at[pl.ds(off, ECH)], nbuf)

        pltpu.sync_copy(relp.at[pl.ds(off, ECH)], rbuf)

        def step(i, c2):
            for s in range(4):
                b = i * 64 + s * 16
                n = nbuf[pl.ds(b, 16)]
                r = rbuf[pl.ds(b, 16)]
                local = n - lo
                mask = jnp.logical_and(n >= lo, n < hi)
                plsc.addupdate_scatter(ctab, [r, local], ones16, mask=mask)
            return c2

        lax.fori_loop(0, ECH // 64, step, 0)
        return carry

    lax.fori_loop(0, NCH, chunk, 0)
    pltpu.sync_copy(ctab, oc.at[cid, sid])


def _table_body(c_ref, h_ref, w_ref, b_ref, tab_ref):
    ch = c_ref[0]
    ct = c_ref[1]
    h3 = h_ref[...]
    sum_h = jnp.sum(
        lax.dot_general(ch, h3, (((2,), (1,)), ((0,), (0,))),
                        preferred_element_type=jnp.float32), axis=0)
    sum_t = jnp.sum(
        lax.dot_general(ct, h3, (((2,), (1,)), ((0,), (0,))),
                        preferred_element_type=jnp.float32), axis=0)
    cnt = jnp.sum(jnp.sum(ch, axis=2, keepdims=True), axis=0)
    denom = jnp.maximum(cnt, 1.0)
    m = jnp.concatenate([sum_h / denom, sum_t / denom], axis=1)
    x2 = lax.dot_general(m, w_ref[...], (((1,), (1,)), ((), ())),
                         preferred_element_type=jnp.float32) + b_ref[...]
    tab_ref[...] = jnp.concatenate([x2, m], axis=1)


def _make_table(oc, h3, W_sr1, b_sr1):
    return pl.pallas_call(
        _table_body,
        out_shape=jax.ShapeDtypeStruct((RP, OUT_W), jnp.float32),
    )(oc, h3, W_sr1, b_sr1.reshape(1, R_HID))


def _out_body(rel_ref, xres_ref, tab_ref, o_ref):
    r = rel_ref[...]
    cols = lax.broadcasted_iota(jnp.int32, (1, RP), 1)
    onehot = jnp.equal(r, cols).astype(jnp.float32)
    look = lax.dot_general(
        onehot, tab_ref[...], (((1,), (0,)), ((), ())),
        preferred_element_type=jnp.float32)
    o_ref[:, 0:T_HID] = xres_ref[...] + look[:, 0:T_HID]
    o_ref[:, T_HID:] = look[:, T_HID:]


def _assemble(rel2, x_res1, table):
    return pl.pallas_call(
        _out_body,
        grid=(E // BLK,),
        in_specs=[
            pl.BlockSpec((BLK, 1), lambda i: (i, 0)),
            pl.BlockSpec((BLK, R_HID), lambda i: (i, 0)),
            pl.BlockSpec((RP, OUT_W), lambda i: (0, 0)),
        ],
        out_specs=pl.BlockSpec((BLK, OUT_W), lambda i: (i, 0)),
        out_shape=jax.ShapeDtypeStruct((E, OUT_W), jnp.float32),
        compiler_params=pltpu.CompilerParams(
            dimension_semantics=("arbitrary",)),
    )(rel2, x_res1, table)


def kernel(x_e, edge_index, rel, x_res1, rel_size, W_tc1, b_tc1, W_sr1,
           b_sr1, a1, a5):
    h = _project(x_e, W_tc1, b_tc1)
    pad = EPAD - E
    headf = jnp.concatenate([edge_index[0], jnp.zeros((pad,), jnp.int32)])
    tailf = jnp.concatenate([edge_index[1], jnp.zeros((pad,), jnp.int32)])
    relf = jnp.concatenate([rel, jnp.full((pad,), NREL, jnp.int32)])
    zc = jnp.zeros((RP, NPT_P), jnp.float32)
    oc = _sc_histogram(headf, tailf, relf, zc)
    h3 = jnp.pad(h.reshape(NSC, NPT, T_HID),
                 ((0, 0), (0, NPT_P - NPT), (0, 0)))
    table = _make_table(oc, h3, W_sr1, b_sr1)
    return _assemble(rel.reshape(E, 1), x_res1, table)


# speedup vs baseline: 2.8476x; 1.0238x over previous
import functools

import jax
import jax.numpy as jnp
from jax import lax
from jax.experimental import pallas as pl
from jax.experimental.pallas import tpu as pltpu
from jax.experimental.pallas import tpu_sc as plsc

N = 10000
E = 320000
E_HID = 128
T_HID = 64
R_HID = 64
NREL = 100
RP = 104
NSC = 16
NPT = 625
NPT_P = 640
ECH = 8192
NCH = E // ECH
HROWS = 2000
BLK = 2560
OUT_W = R_HID + 2 * T_HID


def _h_body(x_ref, w_ref, b_ref, o_ref):
    o_ref[...] = lax.dot_general(
        x_ref[...], w_ref[...], (((1,), (1,)), ((), ())),
        preferred_element_type=jnp.float32) + b_ref[...]


def _project(x_e, W_tc1, b_tc1):
    return pl.pallas_call(
        _h_body,
        grid=(N // HROWS,),
        in_specs=[
            pl.BlockSpec((HROWS, E_HID), lambda i: (i, 0)),
            pl.BlockSpec((T_HID, E_HID), lambda i: (0, 0)),
            pl.BlockSpec((1, T_HID), lambda i: (0, 0)),
        ],
        out_specs=pl.BlockSpec((HROWS, T_HID), lambda i: (i, 0)),
        out_shape=jax.ShapeDtypeStruct((N, T_HID), jnp.float32),
    )(x_e, W_tc1, b_tc1.reshape(1, T_HID))


_SC_MESH = plsc.VectorSubcoreMesh(core_axis_name="c", subcore_axis_name="s")


@functools.partial(
    pl.kernel,
    mesh=_SC_MESH,
    compiler_params=pltpu.CompilerParams(use_tc_tiling_on_sc=False,
                                         needs_layout_passes=False),
    out_type=jax.ShapeDtypeStruct((2, NSC, RP, NPT_P), jnp.float32),
    scratch_types=[
        pltpu.VMEM((ECH,), jnp.int32),
        pltpu.VMEM((ECH,), jnp.int32),
        pltpu.VMEM((RP, NPT_P), jnp.float32),
    ],
)
def _sc_histogram(edge_index, rel, zc, oc, nbuf, rbuf, ctab):
    cid = lax.axis_index("c")
    sid = lax.axis_index("s")
    lo = sid * NPT
    hi = lo + NPT

    ones16 = jnp.full((16,), 1.0, jnp.float32)
    pltpu.sync_copy(zc, ctab)

    def step(i, c2):
        for s in range(4):
            b = i * 64 + s * 16
            n = nbuf[pl.ds(b, 16)]
            r = rbuf[pl.ds(b, 16)]
            local = n - lo
            mask = jnp.logical_and(n >= lo, n < hi)
            plsc.addupdate_scatter(ctab, [r, local], ones16, mask=mask)
        return c2

    def chunk(j, carry):
        off = j * ECH
        pltpu.sync_copy(edge_index.at[cid, pl.ds(off, ECH)], nbuf)
        pltpu.sync_copy(rel.at[pl.ds(off, ECH)], rbuf)
        lax.fori_loop(0, ECH // 64, step, 0)
        return carry

    lax.fori_loop(0, NCH, chunk, 0)

    woff = E - ECH
    pltpu.sync_copy(edge_index.at[cid, pl.ds(woff, ECH)], nbuf)
    pltpu.sync_copy(rel.at[pl.ds(woff, ECH)], rbuf)
    lax.fori_loop((NCH * ECH - woff) // 64, ECH // 64, step, 0)

    pltpu.sync_copy(ctab, oc.at[cid, sid])


def _out_body(rel_ref, xres_ref, c_ref, h_ref, w_ref, b_ref, o_ref, tab_ref):
    @pl.when(pl.program_id(0) == 0)
    def _build_table():
        ch = c_ref[0]
        ct = c_ref[1]
        h3 = h_ref[...]
        sum_h = jnp.sum(
            lax.dot_general(ch, h3, (((2,), (1,)), ((0,), (0,))),
                            preferred_element_type=jnp.float32), axis=0)
        sum_t = jnp.sum(
            lax.dot_general(ct, h3, (((2,), (1,)), ((0,), (0,))),
                            preferred_element_type=jnp.float32), axis=0)
        cnt = jnp.sum(jnp.sum(ch, axis=2, keepdims=True), axis=0)
        denom = jnp.maximum(cnt, 1.0)
        m = jnp.concatenate([sum_h / denom, sum_t / denom], axis=1)
        x2 = lax.dot_general(m, w_ref[...], (((1,), (1,)), ((), ())),
                             preferred_element_type=jnp.float32) + b_ref[...]
        tab_ref[...] = jnp.concatenate([x2, m], axis=1)

    r = rel_ref[...]
    cols = lax.broadcasted_iota(jnp.int32, (1, RP), 1)
    onehot = jnp.equal(r, cols).astype(jnp.float32)
    look = lax.dot_general(
        onehot, tab_ref[...], (((1,), (0,)), ((), ())),
        preferred_element_type=jnp.float32)
    o_ref[:, 0:T_HID] = xres_ref[...] + look[:, 0:T_HID]
    o_ref[:, T_HID:] = look[:, T_HID:]


def _assemble(rel2, x_res1, oc, h3, W_sr1, b_sr1):
    return pl.pallas_call(
        _out_body,
        grid=(E // BLK,),
        in_specs=[
            pl.BlockSpec((BLK, 1), lambda i: (i, 0)),
            pl.BlockSpec((BLK, R_HID), lambda i: (i, 0)),
            pl.BlockSpec((2, NSC, RP, NPT_P), lambda i: (0, 0, 0, 0)),
            pl.BlockSpec((NSC, NPT_P, T_HID), lambda i: (0, 0, 0)),
            pl.BlockSpec((T_HID, 2 * T_HID), lambda i: (0, 0)),
            pl.BlockSpec((1, R_HID), lambda i: (0, 0)),
        ],
        out_specs=pl.BlockSpec((BLK, OUT_W), lambda i: (i, 0)),
        out_shape=jax.ShapeDtypeStruct((E, OUT_W), jnp.float32),
        scratch_shapes=[pltpu.VMEM((RP, OUT_W), jnp.float32)],
        compiler_params=pltpu.CompilerParams(
            dimension_semantics=("arbitrary",)),
    )(rel2, x_res1, oc, h3, W_sr1, b_sr1.reshape(1, R_HID))


def kernel(x_e, edge_index, rel, x_res1, rel_size, W_tc1, b_tc1, W_sr1,
           b_sr1, a1, a5):
    h = _project(x_e, W_tc1, b_tc1)
    zc = jnp.zeros((RP, NPT_P), jnp.float32)
    oc = _sc_histogram(edge_index, rel, zc)
    h3 = jnp.pad(h.reshape(NSC, NPT, T_HID),
                 ((0, 0), (0, NPT_P - NPT), (0, 0)))
    return _assemble(rel.reshape(E, 1), x_res1, oc, h3, W_sr1, b_sr1)


# speedup vs baseline: 2.9109x; 1.0222x over previous
import functools

import jax
import jax.numpy as jnp
from jax import lax
from jax.experimental import pallas as pl
from jax.experimental.pallas import tpu as pltpu
from jax.experimental.pallas import tpu_sc as plsc

N = 10000
E = 320000
E_HID = 128
T_HID = 64
R_HID = 64
NREL = 100
RP = 104
NSC = 16
NPT = 625
NPT_P = 640
ECH = 8192
NCH = E // ECH
HROWS = 2000
BLK = 6400
OUT_W = R_HID + 2 * T_HID


def _h_body(x_ref, w_ref, b_ref, o_ref):
    o_ref[...] = lax.dot_general(
        x_ref[...], w_ref[...], (((1,), (1,)), ((), ())),
        preferred_element_type=jnp.float32) + b_ref[...]


def _project(x_e, W_tc1, b_tc1):
    return pl.pallas_call(
        _h_body,
        grid=(N // HROWS,),
        in_specs=[
            pl.BlockSpec((HROWS, E_HID), lambda i: (i, 0)),
            pl.BlockSpec((T_HID, E_HID), lambda i: (0, 0)),
            pl.BlockSpec((1, T_HID), lambda i: (0, 0)),
        ],
        out_specs=pl.BlockSpec((HROWS, T_HID), lambda i: (i, 0)),
        out_shape=jax.ShapeDtypeStruct((N, T_HID), jnp.float32),
    )(x_e, W_tc1, b_tc1.reshape(1, T_HID))


_SC_MESH = plsc.VectorSubcoreMesh(core_axis_name="c", subcore_axis_name="s")


@functools.partial(
    pl.kernel,
    mesh=_SC_MESH,
    compiler_params=pltpu.CompilerParams(use_tc_tiling_on_sc=False,
                                         needs_layout_passes=False),
    out_type=jax.ShapeDtypeStruct((2, NSC, RP, NPT_P), jnp.float32),
    scratch_types=[
        pltpu.VMEM((ECH,), jnp.int32),
        pltpu.VMEM((ECH,), jnp.int32),
        pltpu.VMEM((RP, NPT_P), jnp.float32),
    ],
)
def _sc_histogram(edge_index, rel, zc, oc, nbuf, rbuf, ctab):
    cid = lax.axis_index("c")
    sid = lax.axis_index("s")
    lo = sid * NPT
    hi = lo + NPT

    ones16 = jnp.full((16,), 1.0, jnp.float32)
    pltpu.sync_copy(zc, ctab)

    def step(i, c2):
        for s in range(4):
            b = i * 64 + s * 16
            n = nbuf[pl.ds(b, 16)]
            r = rbuf[pl.ds(b, 16)]
            local = n - lo
            mask = jnp.logical_and(n >= lo, n < hi)
            plsc.addupdate_scatter(ctab, [r, local], ones16, mask=mask)
        return c2

    def chunk(j, carry):
        off = j * ECH
        pltpu.sync_copy(edge_index.at[cid, pl.ds(off, ECH)], nbuf)
        pltpu.sync_copy(rel.at[pl.ds(off, ECH)], rbuf)
        lax.fori_loop(0, ECH // 64, step, 0)
        return carry

    lax.fori_loop(0, NCH, chunk, 0)

    woff = E - ECH
    pltpu.sync_copy(edge_index.at[cid, pl.ds(woff, ECH)], nbuf)
    pltpu.sync_copy(rel.at[pl.ds(woff, ECH)], rbuf)
    lax.fori_loop((NCH * ECH - woff) // 64, ECH // 64, step, 0)

    pltpu.sync_copy(ctab, oc.at[cid, sid])


def _out_body(rel_ref, xres_ref, c_ref, h_ref, w_ref, b_ref, o_ref, tab_ref):
    @pl.when(pl.program_id(0) == 0)
    def _build_table():
        ch = c_ref[0]
        ct = c_ref[1]
        h3 = h_ref[...]
        sum_h = jnp.sum(
            lax.dot_general(ch, h3, (((2,), (1,)), ((0,), (0,))),
                            preferred_element_type=jnp.float32), axis=0)
        sum_t = jnp.sum(
            lax.dot_general(ct, h3, (((2,), (1,)), ((0,), (0,))),
                            preferred_element_type=jnp.float32), axis=0)
        cnt = jnp.sum(jnp.sum(ch, axis=2, keepdims=True), axis=0)
        denom = jnp.maximum(cnt, 1.0)
        m = jnp.concatenate([sum_h / denom, sum_t / denom], axis=1)
        x2 = lax.dot_general(m, w_ref[...], (((1,), (1,)), ((), ())),
                             preferred_element_type=jnp.float32) + b_ref[...]
        tab_ref[...] = jnp.concatenate([x2, m], axis=1)

    r = rel_ref[...]
    cols = lax.broadcasted_iota(jnp.int32, (1, RP), 1)
    onehot = jnp.equal(r, cols).astype(jnp.float32)
    look = lax.dot_general(
        onehot, tab_ref[...], (((1,), (0,)), ((), ())),
        preferred_element_type=jnp.float32)
    o_ref[:, 0:T_HID] = xres_ref[...] + look[:, 0:T_HID]
    o_ref[:, T_HID:] = look[:, T_HID:]


def _assemble(rel2, x_res1, oc, h3, W_sr1, b_sr1):
    return pl.pallas_call(
        _out_body,
        grid=(E // BLK,),
        in_specs=[
            pl.BlockSpec((BLK, 1), lambda i: (i, 0)),
            pl.BlockSpec((BLK, R_HID), lambda i: (i, 0)),
            pl.BlockSpec((2, NSC, RP, NPT_P), lambda i: (0, 0, 0, 0)),
            pl.BlockSpec((NSC, NPT_P, T_HID), lambda i: (0, 0, 0)),
            pl.BlockSpec((T_HID, 2 * T_HID), lambda i: (0, 0)),
            pl.BlockSpec((1, R_HID), lambda i: (0, 0)),
        ],
        out_specs=pl.BlockSpec((BLK, OUT_W), lambda i: (i, 0)),
        out_shape=jax.ShapeDtypeStruct((E, OUT_W), jnp.float32),
        scratch_shapes=[pltpu.VMEM((RP, OUT_W), jnp.float32)],
        compiler_params=pltpu.CompilerParams(
            dimension_semantics=("arbitrary",)),
    )(rel2, x_res1, oc, h3, W_sr1, b_sr1.reshape(1, R_HID))


def kernel(x_e, edge_index, rel, x_res1, rel_size, W_tc1, b_tc1, W_sr1,
           b_sr1, a1, a5):
    h = _project(x_e, W_tc1, b_tc1)
    zc = jnp.zeros((RP, NPT_P), jnp.float32)
    oc = _sc_histogram(edge_index, rel, zc)
    h3 = jnp.pad(h.reshape(NSC, NPT, T_HID),
                 ((0, 0), (0, NPT_P - NPT), (0, 0)))
    return _assemble(rel.reshape(E, 1), x_res1, oc, h3, W_sr1, b_sr1)


# speedup vs baseline: 3.1552x; 1.0839x over previous
import functools

import jax
import jax.numpy as jnp
from jax import lax
from jax.experimental import pallas as pl
from jax.experimental.pallas import tpu as pltpu
from jax.experimental.pallas import tpu_sc as plsc

N = 10000
E = 320000
E_HID = 128
T_HID = 64
R_HID = 64
NREL = 100
RP = 104
NSC = 16
NPT = 625
NPT_P = 640
ECH = 8192
NCH = E // ECH
HROWS = 2000
BLK = 6400
OUT_W = R_HID + 2 * T_HID


def _h_body(x_ref, w_ref, b_ref, o_ref):
    o_ref[...] = lax.dot_general(
        x_ref[...], w_ref[...], (((1,), (1,)), ((), ())),
        preferred_element_type=jnp.float32) + b_ref[...]


def _project(x_e, W_tc1, b_tc1):
    return pl.pallas_call(
        _h_body,
        grid=(N // HROWS,),
        in_specs=[
            pl.BlockSpec((HROWS, E_HID), lambda i: (i, 0)),
            pl.BlockSpec((T_HID, E_HID), lambda i: (0, 0)),
            pl.BlockSpec((1, T_HID), lambda i: (0, 0)),
        ],
        out_specs=pl.BlockSpec((HROWS, T_HID), lambda i: (i, 0)),
        out_shape=jax.ShapeDtypeStruct((N, T_HID), jnp.float32),
    )(x_e, W_tc1, b_tc1.reshape(1, T_HID))


_SC_MESH = plsc.VectorSubcoreMesh(core_axis_name="c", subcore_axis_name="s")


@functools.partial(
    pl.kernel,
    mesh=_SC_MESH,
    compiler_params=pltpu.CompilerParams(use_tc_tiling_on_sc=False,
                                         needs_layout_passes=False),
    out_type=jax.ShapeDtypeStruct((2, NSC, RP, NPT_P), jnp.float32),
    scratch_types=[
        pltpu.VMEM((ECH,), jnp.int32),
        pltpu.VMEM((ECH,), jnp.int32),
        pltpu.VMEM((RP, NPT_P), jnp.float32),
    ],
)
def _sc_histogram(edge_index, rel, zc, oc, nbuf, rbuf, ctab):
    cid = lax.axis_index("c")
    sid = lax.axis_index("s")
    lo = sid * NPT
    hi = lo + NPT

    ones16 = jnp.full((16,), 1.0, jnp.float32)
    pltpu.sync_copy(zc, ctab)

    def step(i, c2):
        for s in range(4):
            b = i * 64 + s * 16
            n = nbuf[pl.ds(b, 16)]
            r = rbuf[pl.ds(b, 16)]
            local = n - lo
            mask = jnp.logical_and(n >= lo, n < hi)
            plsc.addupdate_scatter(ctab, [r, local], ones16, mask=mask)
        return c2

    def chunk(j, carry):
        off = j * ECH
        pltpu.sync_copy(edge_index.at[cid, pl.ds(off, ECH)], nbuf)
        pltpu.sync_copy(rel.at[pl.ds(off, ECH)], rbuf)
        lax.fori_loop(0, ECH // 64, step, 0)
        return carry

    lax.fori_loop(0, NCH, chunk, 0)

    woff = E - ECH
    pltpu.sync_copy(edge_index.at[cid, pl.ds(woff, ECH)], nbuf)
    pltpu.sync_copy(rel.at[pl.ds(woff, ECH)], rbuf)
    lax.fori_loop((NCH * ECH - woff) // 64, ECH // 64, step, 0)

    pltpu.sync_copy(ctab, oc.at[cid, sid])


def _out_body(rel_ref, xres_ref, c_ref, h_ref, w_ref, b_ref, o_ref, tab_ref):
    @pl.when(pl.program_id(0) == 0)
    def _build_table():
        ch = c_ref[0]
        ct = c_ref[1]
        h3 = h_ref[...]
        sum_h = jnp.sum(
            lax.dot_general(ch, h3, (((2,), (1,)), ((0,), (0,))),
                            preferred_element_type=jnp.float32), axis=0)
        sum_t = jnp.sum(
            lax.dot_general(ct, h3, (((2,), (1,)), ((0,), (0,))),
                            preferred_element_type=jnp.float32), axis=0)
        cnt = jnp.sum(jnp.sum(ch, axis=2, keepdims=True), axis=0)
        denom = jnp.maximum(cnt, 1.0)
        m = jnp.concatenate([sum_h / denom, sum_t / denom], axis=1)
        x2 = lax.dot_general(m, w_ref[...], (((1,), (1,)), ((), ())),
                             preferred_element_type=jnp.float32) + b_ref[...]
        tab_ref[...] = jnp.concatenate([x2, m], axis=1)

    r = rel_ref[...]
    rows = lax.broadcasted_iota(jnp.int32, (RP, 1), 0)
    onehot_t = jnp.equal(rows, r).astype(jnp.float32)
    look = lax.dot_general(
        onehot_t, tab_ref[...], (((0,), (0,)), ((), ())),
        preferred_element_type=jnp.float32)
    o_ref[:, 0:T_HID] = xres_ref[...] + look[:, 0:T_HID]
    o_ref[:, T_HID:] = look[:, T_HID:]


def _assemble(rel2, x_res1, oc, h3, W_sr1, b_sr1):
    return pl.pallas_call(
        _out_body,
        grid=(E // BLK,),
        in_specs=[
            pl.BlockSpec((1, BLK), lambda i: (0, i)),
            pl.BlockSpec((BLK, R_HID), lambda i: (i, 0)),
            pl.BlockSpec((2, NSC, RP, NPT_P), lambda i: (0, 0, 0, 0)),
            pl.BlockSpec((NSC, NPT_P, T_HID), lambda i: (0, 0, 0)),
            pl.BlockSpec((T_HID, 2 * T_HID), lambda i: (0, 0)),
            pl.BlockSpec((1, R_HID), lambda i: (0, 0)),
        ],
        out_specs=pl.BlockSpec((BLK, OUT_W), lambda i: (i, 0)),
        out_shape=jax.ShapeDtypeStruct((E, OUT_W), jnp.float32),
        scratch_shapes=[pltpu.VMEM((RP, OUT_W), jnp.float32)],
        compiler_params=pltpu.CompilerParams(
            dimension_semantics=("arbitrary",)),
    )(rel2, x_res1, oc, h3, W_sr1, b_sr1.reshape(1, R_HID))


def kernel(x_e, edge_index, rel, x_res1, rel_size, W_tc1, b_tc1, W_sr1,
           b_sr1, a1, a5):
    h = _project(x_e, W_tc1, b_tc1)
    zc = jnp.zeros((RP, NPT_P), jnp.float32)
    oc = _sc_histogram(edge_index, rel, zc)
    h3 = jnp.pad(h.reshape(NSC, NPT, T_HID),
                 ((0, 0), (0, NPT_P - NPT), (0, 0)))
    return _assemble(rel.reshape(1, E), x_res1, oc, h3, W_sr1, b_sr1)
